# baseline (speedup 1.0000x reference)
import functools

import jax
import jax.numpy as jnp
import numpy as np
from jax import lax
from jax.experimental import pallas as pl
from jax.experimental.pallas import tpu as pltpu
from jax.experimental.pallas import tpu_sc as plsc

_MASK_FREQ = 0.15
_TOTAL_TOKENS = 32768
_D_FEAT = 512


def _rotl32(x, r):
    return ((x << np.uint32(r)) | (x >> np.uint32(32 - r))).astype(np.uint32)


def _threefry2x32(k0, k1, x0, x1):
    rotations = ((13, 15, 26, 6), (17, 29, 16, 24))
    ks = (
        np.uint32(k0),
        np.uint32(k1),
        np.uint32(k0) ^ np.uint32(k1) ^ np.uint32(0x1BD11BDA),
    )
    x0 = (x0 + ks[0]).astype(np.uint32)
    x1 = (x1 + ks[1]).astype(np.uint32)
    for d in range(5):
        for rot in rotations[d % 2]:
            x0 = (x0 + x1).astype(np.uint32)
            x1 = _rotl32(x1, rot)
            x1 = (x1 ^ x0).astype(np.uint32)
        x0 = (x0 + ks[(d + 1) % 3]).astype(np.uint32)
        x1 = (x1 + ks[(d + 2) % 3] + np.uint32(d + 1)).astype(np.uint32)
    return x0, x1


def _uniform_key42(n):
    lo = np.arange(n, dtype=np.uint32)
    hi = np.zeros(n, np.uint32)
    x0, x1 = _threefry2x32(0, 42, hi, lo)
    bits = (x0 ^ x1).astype(np.uint32)
    return ((bits >> np.uint32(9)) | np.uint32(0x3F800000)).view(np.float32) - np.float32(1.0)


_KEPT = _uniform_key42(_TOTAL_TOKENS) > _MASK_FREQ
_N_KEPT_ROWS = int(_KEPT.sum())
_IDX = np.nonzero(_KEPT)[0].astype(np.int32)

_NW = 32
_NC = 2
_CH = 48
_NCHUNKS = -(-_N_KEPT_ROWS // _CH)
_TAIL = _N_KEPT_ROWS - (_NCHUNKS - 1) * _CH
_SLOTS = -(-_NCHUNKS // _NW)
_TAIL_W = (_NCHUNKS - 1) % _NW

_IDX_PAD = np.full((_NW * _SLOTS * _CH,), _IDX[-1], np.int32)
_IDX_PAD[:_N_KEPT_ROWS] = _IDX
_IDX_WORKER = np.full((_NW, _SLOTS * _CH), _IDX[-1], np.int32)
for _w in range(_NW):
    for _s in range(_SLOTS):
        _c = _w + _s * _NW
        if _c < _NCHUNKS:
            _IDX_WORKER[_w, _s * _CH:(_s + 1) * _CH] = _IDX_PAD[_c * _CH:(_c + 1) * _CH]
_IDX_WORKER = _IDX_WORKER.reshape(-1)

_DST_TAIL = np.minimum(
    np.arange((_NCHUNKS - 1) * _CH, _NCHUNKS * _CH), _N_KEPT_ROWS - 1
).astype(np.int32)


@functools.cache
def _build_sc_gather():
    mesh = plsc.VectorSubcoreMesh(core_axis_name="c", subcore_axis_name="s")

    @functools.partial(
        pl.kernel,
        mesh=mesh,
        out_type=jax.ShapeDtypeStruct((_N_KEPT_ROWS, _D_FEAT), jnp.float32),
        scratch_types=[
            pltpu.VMEM((_SLOTS * _CH,), jnp.int32),
            pltpu.VMEM((_CH,), jnp.int32),
            pltpu.VMEM((_CH, _D_FEAT), jnp.float32),
            pltpu.VMEM((_CH, _D_FEAT), jnp.float32),
            pltpu.VMEM((_CH, _D_FEAT), jnp.float32),
            pltpu.SemaphoreType.DMA,
            pltpu.SemaphoreType.DMA,
            pltpu.SemaphoreType.DMA,
            pltpu.SemaphoreType.DMA,
            pltpu.SemaphoreType.DMA,
            pltpu.SemaphoreType.DMA,
        ],
    )
    def _sc_gather(feat_hbm, idx_hbm, dst_hbm, out_hbm,
                   idx_v, dst_v, buf0, buf1, buf2, g0, g1, g2, s0, s1, s2):
        w = lax.axis_index("s") * _NC + lax.axis_index("c")
        bufs = (buf0, buf1, buf2)
        gsems = (g0, g1, g2)
        ssems = (s0, s1, s2)

        pltpu.sync_copy(
            idx_hbm.at[pl.ds(w * (_SLOTS * _CH), _SLOTS * _CH)], idx_v
        )
        pltpu.sync_copy(dst_hbm, dst_v)

        def start_gather(s):
            b = s % 3

            def issue():
                pltpu.async_copy(
                    feat_hbm.at[idx_v.at[pl.ds(s * _CH, _CH)]], bufs[b], gsems[b]
                )

            if s < _SLOTS - 1:
                issue()
            else:
                pl.when(w <= _TAIL_W)(issue)

        def wait_gather(s):
            b = s % 3
            pltpu.make_async_copy(
                feat_hbm.at[idx_v.at[pl.ds(s * _CH, _CH)]], bufs[b], gsems[b]
            ).wait()

        def start_store(s):
            b = s % 3
            if s < _SLOTS - 1:
                pltpu.async_copy(
                    bufs[b], out_hbm.at[pl.ds((w + s * _NW) * _CH, _CH)], ssems[b]
                )
            else:
                @pl.when(w < _TAIL_W)
                def _():
                    pltpu.async_copy(
                        bufs[b], out_hbm.at[pl.ds((w + s * _NW) * _CH, _CH)],
                        ssems[b],
                    )

                @pl.when(w == _TAIL_W)
                def _():
                    pltpu.async_copy(bufs[b], out_hbm.at[dst_v], ssems[b])

        def wait_store(s):
            b = s % 3
            if s < _SLOTS - 1:
                pltpu.make_async_copy(
                    bufs[b], out_hbm.at[pl.ds((w + s * _NW) * _CH, _CH)], ssems[b]
                ).wait()
            else:
                @pl.when(w < _TAIL_W)
                def _():
                    pltpu.make_async_copy(
                        bufs[b], out_hbm.at[pl.ds((w + s * _NW) * _CH, _CH)],
                        ssems[b],
                    ).wait()

                @pl.when(w == _TAIL_W)
                def _():
                    pltpu.make_async_copy(
                        bufs[b], out_hbm.at[dst_v], ssems[b]
                    ).wait()

        def finish_chunk(s):
            def run():
                wait_gather(s)
                start_store(s)

            if s < _SLOTS - 1:
                run()
            else:
                pl.when(w <= _TAIL_W)(run)

        start_gather(0)
        start_gather(1)
        for s in range(_SLOTS):
            if s + 2 < _SLOTS:
                if s - 1 >= 0:
                    wait_store(s - 1)
                start_gather(s + 2)
            finish_chunk(s)
        for s in range(_SLOTS - 3, _SLOTS):
            wait_store(s)

    return _sc_gather


def kernel(feature):
    return _build_sc_gather()(
        feature, jnp.asarray(_IDX_WORKER), jnp.asarray(_DST_TAIL)
    )
